# baseline (speedup 1.0000x reference)
import functools

import jax
import jax.numpy as jnp
from jax import lax
from jax.experimental import pallas as pl
from jax.experimental.pallas import tpu as pltpu
from jax.experimental.pallas import tpu_sc as plsc

N_NODES = 10000
N_EDGES = 320000
H = 128
BN_EPS = 1e-5

NUM_WORKERS = 32
NSPLIT = 2
E_HALF = N_EDGES // NSPLIT
EDGES_PER_W = E_HALF // NUM_WORKERS
CHUNK = 40
EDGE_TILE = 16000
TILES_PER_HALF = E_HALF // EDGE_TILE


def _gelu(x):
    return 0.5 * x * (1.0 + lax.erf(x * 0.7071067811865476))


def _node_proj_body(hv_ref, wa_ref, wc_ref, b_ref, a_ref, c_ref):
    hv = hv_ref[...]
    a_ref[...] = (
        jnp.dot(hv, wa_ref[...], preferred_element_type=jnp.float32) + b_ref[...]
    )
    c_ref[...] = jnp.dot(hv, wc_ref[...], preferred_element_type=jnp.float32)


def _node_proj(h_V, wa_t, wc_t, b11):
    return pl.pallas_call(
        _node_proj_body,
        out_shape=(
            jax.ShapeDtypeStruct((N_NODES, H), jnp.float32),
            jax.ShapeDtypeStruct((N_NODES, H), jnp.float32),
        ),
    )(h_V, wa_t, wc_t, b11)


NBANK = 3
CHUNKS_PER_W = EDGES_PER_W // CHUNK
HP = H // 2


def _gather_body(edge_base, a_hbm, b_hbm, src_hbm, dst_hbm, g_hbm,
                 idxs_v, idxd_v, rA0, rB0, rA1, rB1, rA2, rB2,
                 sg0, sg1, sg2, sw0, sw1, sw2):
    c = lax.axis_index("c")
    s = lax.axis_index("s")
    wid = s * 2 + c
    base = wid * EDGES_PER_W

    pltpu.sync_copy(src_hbm.at[pl.ds(edge_base + base, EDGES_PER_W)], idxs_v)
    pltpu.sync_copy(dst_hbm.at[pl.ds(edge_base + base, EDGES_PER_W)], idxd_v)

    banks = ((rA0, rB0, sg0, sw0), (rA1, rB1, sg1, sw1), (rA2, rB2, sg2, sw2))
    dummy = g_hbm.at[pl.ds(0, CHUNK)]

    def issue_gather(chunk, bank_i):
        rA, rB, sg, _ = banks[bank_i]
        off = pl.multiple_of(chunk * CHUNK, 8)
        pltpu.async_copy(a_hbm.at[idxs_v.at[pl.ds(off, CHUNK)]], rA, sg)
        pltpu.async_copy(b_hbm.at[idxd_v.at[pl.ds(off, CHUNK)]], rB, sg)

    issue_gather(0, 0)

    def make_branch(bank_i):
        rA, rB, sg, sw = banks[bank_i]
        nbank_i = (bank_i + 1) % NBANK
        _, _, _, sw_n = banks[nbank_i]

        def branch(g):
            @pl.when(g >= 2)
            def _():
                pltpu.make_async_copy(dummy, banks[nbank_i][0], sw_n).wait()

            @pl.when(g < CHUNKS_PER_W - 1)
            def _():
                issue_gather(g + 1, nbank_i)

            pltpu.make_async_copy(dummy, rA, sg).wait()
            pltpu.make_async_copy(dummy, rB, sg).wait()

            def row(r, carry):
                for k in range(H // 16):
                    sl = pl.ds(k * 16, 16)
                    rA[r, sl] = rA[r, sl] + rB[r, sl]
                return carry

            lax.fori_loop(0, CHUNK, row, 0)
            pltpu.async_copy(rA, g_hbm.at[pl.ds(base + g * CHUNK, CHUNK)], sw)

        return branch

    brs = [make_branch(i) for i in range(NBANK)]

    def step(g, carry):
        lax.switch(lax.rem(g, NBANK), brs, g)
        return carry

    lax.fori_loop(0, CHUNKS_PER_W, step, 0)

    for chunk in (CHUNKS_PER_W - 2, CHUNKS_PER_W - 1):
        rA, _, _, sw = banks[chunk % NBANK]
        pltpu.make_async_copy(dummy, rA, sw).wait()


def _sc_gather(a, b, src, dst, edge_base):
    mesh = plsc.VectorSubcoreMesh(
        core_axis_name="c", subcore_axis_name="s", num_cores=2, num_subcores=16
    )
    rows = pltpu.VMEM((CHUNK, H), jnp.float32)
    return pl.kernel(
        functools.partial(_gather_body, edge_base),
        out_type=jax.ShapeDtypeStruct((E_HALF, H), jnp.float32),
        mesh=mesh,
        scratch_types=[
            pltpu.VMEM((EDGES_PER_W,), jnp.int32),
            pltpu.VMEM((EDGES_PER_W,), jnp.int32),
            rows, rows, rows, rows, rows, rows,
            pltpu.SemaphoreType.DMA, pltpu.SemaphoreType.DMA,
            pltpu.SemaphoreType.DMA, pltpu.SemaphoreType.DMA,
            pltpu.SemaphoreType.DMA, pltpu.SemaphoreType.DMA,
        ],
    )(a, b, src, dst)


def _mlp_body(he_ref, g_ref, w1_ref, w2_ref, b2_ref, w3_ref, b3_ref,
              x_ref, s_ref, s2_ref):
    he = he_ref[...]
    pre1 = (
        jnp.dot(
            he.astype(jnp.bfloat16), w1_ref[...],
            preferred_element_type=jnp.float32,
        )
        + g_ref[...]
    )
    h1 = _gelu(pre1)
    h2 = _gelu(
        jnp.dot(
            h1.astype(jnp.bfloat16), w2_ref[...],
            preferred_element_type=jnp.float32,
        )
        + b2_ref[...]
    )
    msg = (
        jnp.dot(
            h2.astype(jnp.bfloat16), w3_ref[...],
            preferred_element_type=jnp.float32,
        )
        + b3_ref[...]
    )
    x = he + msg
    x_ref[...] = x.astype(jnp.bfloat16)

    xr = x.reshape(EDGE_TILE // 8, 8, H)
    ps = jnp.sum(xr, axis=0)
    ps2 = jnp.sum(xr * xr, axis=0)

    @pl.when(pl.program_id(0) == 0)
    def _init():
        s_ref[...] = jnp.zeros_like(s_ref)
        s2_ref[...] = jnp.zeros_like(s2_ref)

    s_ref[...] += ps
    s2_ref[...] += ps2


def _mlp_body_alias(he_ref, g_ref, xp_ref, w1_ref, w2_ref, b2_ref, w3_ref,
                    b3_ref, x_ref, s_ref, s2_ref):
    del xp_ref
    _mlp_body(he_ref, g_ref, w1_ref, w2_ref, b2_ref, w3_ref, b3_ref,
              x_ref, s_ref, s2_ref)


def _mlp_pass(h_E, g, x_prev, w1_t, w2_t, b12, w3_t, b13, half):
    off = half * TILES_PER_HALF
    he_spec = pl.BlockSpec((EDGE_TILE, H), lambda i: (i + off, 0))
    g_spec = pl.BlockSpec((EDGE_TILE, H), lambda i: (i, 0))
    full = pl.BlockSpec((H, H), lambda i: (0, 0))
    vec = pl.BlockSpec((1, H), lambda i: (0, 0))
    stat_spec = pl.BlockSpec((8, H), lambda i: (0, 0))
    out_shape = (
        jax.ShapeDtypeStruct((N_EDGES, H), jnp.bfloat16),
        jax.ShapeDtypeStruct((8, H), jnp.float32),
        jax.ShapeDtypeStruct((8, H), jnp.float32),
    )
    out_specs = (
        pl.BlockSpec((EDGE_TILE, H), lambda i: (i + off, 0)),
        stat_spec,
        stat_spec,
    )
    if x_prev is None:
        return pl.pallas_call(
            _mlp_body,
            grid=(TILES_PER_HALF,),
            in_specs=[he_spec, g_spec, full, full, vec, full, vec],
            out_specs=out_specs,
            out_shape=out_shape,
        )(h_E, g, w1_t, w2_t, b12, w3_t, b13)
    return pl.pallas_call(
        _mlp_body_alias,
        grid=(TILES_PER_HALF,),
        in_specs=[
            he_spec, g_spec,
            pl.BlockSpec(memory_space=pltpu.MemorySpace.HBM),
            full, full, vec, full, vec,
        ],
        out_specs=out_specs,
        out_shape=out_shape,
        input_output_aliases={2: 0},
    )(h_E, g, x_prev, w1_t, w2_t, b12, w3_t, b13)


def _bn_body(x_ref, sa_ref, s2a_ref, sb_ref, s2b_ref, g_ref, be_ref, o_ref):
    s = jnp.sum(sa_ref[...] + sb_ref[...], axis=0, keepdims=True)
    s2 = jnp.sum(s2a_ref[...] + s2b_ref[...], axis=0, keepdims=True)
    inv_n = 1.0 / N_EDGES
    mean = s * inv_n
    var = s2 * inv_n - mean * mean
    inv = lax.rsqrt(var + BN_EPS)
    scale = g_ref[...] * inv
    shift = be_ref[...] - mean * scale
    o_ref[...] = x_ref[...].astype(jnp.float32) * scale + shift


def _bn_apply(x, sa, s2a, sb, s2b, gamma, beta):
    n_tiles = N_EDGES // EDGE_TILE
    edge_spec = pl.BlockSpec((EDGE_TILE, H), lambda i: (i, 0))
    stat_spec = pl.BlockSpec((8, H), lambda i: (0, 0))
    return pl.pallas_call(
        _bn_body,
        grid=(n_tiles,),
        in_specs=[
            edge_spec,
            stat_spec, stat_spec, stat_spec, stat_spec,
            pl.BlockSpec((1, H), lambda i: (0, 0)),
            pl.BlockSpec((1, H), lambda i: (0, 0)),
        ],
        out_specs=edge_spec,
        out_shape=jax.ShapeDtypeStruct((N_EDGES, H), jnp.float32),
    )(x, sa, s2a, sb, s2b, gamma, beta)


def kernel(h_V, h_E, edge_idx, batch_id, W11_w, W11_b, W12_w, W12_b,
           W13_w, W13_b, bn_gamma, bn_beta):
    del batch_id
    src = edge_idx[0].astype(jnp.int32)
    dst = edge_idx[1].astype(jnp.int32)

    wa_t = W11_w[:, :H].T
    w1_t = W11_w[:, H:2 * H].T.astype(jnp.bfloat16)
    wc_t = W11_w[:, 2 * H:].T
    b11 = W11_b.reshape(1, H)
    w2_t = W12_w.T.astype(jnp.bfloat16)
    w3_t = W13_w.T.astype(jnp.bfloat16)
    b12 = W12_b.reshape(1, H)
    b13 = W13_b.reshape(1, H)
    gamma = bn_gamma.reshape(1, H)
    beta = bn_beta.reshape(1, H)

    a, b = _node_proj(h_V, wa_t, wc_t, b11)
    g0 = _sc_gather(a, b, src, dst, 0)
    g1 = _sc_gather(a, b, src, dst, E_HALF)
    x0, sa, s2a = _mlp_pass(h_E, g0, None, w1_t, w2_t, b12, w3_t, b13, 0)
    x, sb, s2b = _mlp_pass(h_E, g1, x0, w1_t, w2_t, b12, w3_t, b13, 1)
    return _bn_apply(x, sa, s2a, sb, s2b, gamma, beta)


# speedup vs baseline: 1.0436x; 1.0436x over previous
import functools

import jax
import jax.numpy as jnp
from jax import lax
from jax.experimental import pallas as pl
from jax.experimental.pallas import tpu as pltpu
from jax.experimental.pallas import tpu_sc as plsc

N_NODES = 10000
N_EDGES = 320000
H = 128
BN_EPS = 1e-5

NUM_WORKERS = 32
NSPLIT = 5
E_SPLIT = N_EDGES // NSPLIT
EDGES_PER_W = E_SPLIT // NUM_WORKERS
CHUNK = 80
EDGE_TILE = 16000
TILES_PER_SPLIT = E_SPLIT // EDGE_TILE


def _gelu(x):
    return 0.5 * x * (1.0 + lax.erf(x * 0.7071067811865476))


def _node_proj_body(hv_ref, wa_ref, wc_ref, b_ref, a_ref, c_ref):
    hv = hv_ref[...]
    a_ref[...] = (
        jnp.dot(hv, wa_ref[...], preferred_element_type=jnp.float32) + b_ref[...]
    )
    c_ref[...] = jnp.dot(hv, wc_ref[...], preferred_element_type=jnp.float32)


def _node_proj(h_V, wa_t, wc_t, b11):
    return pl.pallas_call(
        _node_proj_body,
        out_shape=(
            jax.ShapeDtypeStruct((N_NODES, H), jnp.float32),
            jax.ShapeDtypeStruct((N_NODES, H), jnp.float32),
        ),
    )(h_V, wa_t, wc_t, b11)


NBANK = 3
CHUNKS_PER_W = EDGES_PER_W // CHUNK
HP = H // 2


def _gather_body(edge_base, a_hbm, b_hbm, src_hbm, dst_hbm, g_hbm,
                 idxs_v, idxd_v, rA0, rB0, rA1, rB1, rA2, rB2,
                 sg0, sg1, sg2, sw0, sw1, sw2):
    c = lax.axis_index("c")
    s = lax.axis_index("s")
    wid = s * 2 + c
    base = wid * EDGES_PER_W

    pltpu.sync_copy(src_hbm.at[pl.ds(edge_base + base, EDGES_PER_W)], idxs_v)
    pltpu.sync_copy(dst_hbm.at[pl.ds(edge_base + base, EDGES_PER_W)], idxd_v)

    banks = ((rA0, rB0, sg0, sw0), (rA1, rB1, sg1, sw1), (rA2, rB2, sg2, sw2))
    dummy = g_hbm.at[pl.ds(0, CHUNK)]

    def issue_gather(chunk, bank_i):
        rA, rB, sg, _ = banks[bank_i]
        off = pl.multiple_of(chunk * CHUNK, 8)
        pltpu.async_copy(a_hbm.at[idxs_v.at[pl.ds(off, CHUNK)]], rA, sg)
        pltpu.async_copy(b_hbm.at[idxd_v.at[pl.ds(off, CHUNK)]], rB, sg)

    issue_gather(0, 0)

    def make_branch(bank_i):
        rA, rB, sg, sw = banks[bank_i]
        nbank_i = (bank_i + 1) % NBANK
        _, _, _, sw_n = banks[nbank_i]

        def branch(g):
            @pl.when(g >= 2)
            def _():
                pltpu.make_async_copy(dummy, banks[nbank_i][0], sw_n).wait()

            @pl.when(g < CHUNKS_PER_W - 1)
            def _():
                issue_gather(g + 1, nbank_i)

            pltpu.make_async_copy(dummy, rA, sg).wait()
            pltpu.make_async_copy(dummy, rB, sg).wait()

            def row(r, carry):
                for k in range(H // 16):
                    sl = pl.ds(k * 16, 16)
                    rA[r, sl] = rA[r, sl] + rB[r, sl]
                return carry

            lax.fori_loop(0, CHUNK, row, 0)
            pltpu.async_copy(rA, g_hbm.at[pl.ds(base + g * CHUNK, CHUNK)], sw)

        return branch

    brs = [make_branch(i) for i in range(NBANK)]

    def step(g, carry):
        lax.switch(lax.rem(g, NBANK), brs, g)
        return carry

    lax.fori_loop(0, CHUNKS_PER_W, step, 0)

    for chunk in (CHUNKS_PER_W - 2, CHUNKS_PER_W - 1):
        rA, _, _, sw = banks[chunk % NBANK]
        pltpu.make_async_copy(dummy, rA, sw).wait()


def _sc_gather(a, b, src, dst, edge_base):
    mesh = plsc.VectorSubcoreMesh(
        core_axis_name="c", subcore_axis_name="s", num_cores=2, num_subcores=16
    )
    rows = pltpu.VMEM((CHUNK, H), jnp.float32)
    return pl.kernel(
        functools.partial(_gather_body, edge_base),
        out_type=jax.ShapeDtypeStruct((E_SPLIT, H), jnp.float32),
        mesh=mesh,
        scratch_types=[
            pltpu.VMEM((EDGES_PER_W,), jnp.int32),
            pltpu.VMEM((EDGES_PER_W,), jnp.int32),
            rows, rows, rows, rows, rows, rows,
            pltpu.SemaphoreType.DMA, pltpu.SemaphoreType.DMA,
            pltpu.SemaphoreType.DMA, pltpu.SemaphoreType.DMA,
            pltpu.SemaphoreType.DMA, pltpu.SemaphoreType.DMA,
        ],
    )(a, b, src, dst)


def _mlp_body(he_ref, g_ref, w1_ref, w2_ref, b2_ref, w3_ref, b3_ref,
              x_ref, s_ref, s2_ref):
    he = he_ref[...]
    pre1 = (
        jnp.dot(
            he.astype(jnp.bfloat16), w1_ref[...],
            preferred_element_type=jnp.float32,
        )
        + g_ref[...]
    )
    h1 = _gelu(pre1)
    h2 = _gelu(
        jnp.dot(
            h1.astype(jnp.bfloat16), w2_ref[...],
            preferred_element_type=jnp.float32,
        )
        + b2_ref[...]
    )
    msg = (
        jnp.dot(
            h2.astype(jnp.bfloat16), w3_ref[...],
            preferred_element_type=jnp.float32,
        )
        + b3_ref[...]
    )
    x = he + msg
    x_ref[...] = x.astype(jnp.bfloat16)

    xr = x.reshape(EDGE_TILE // 8, 8, H)
    ps = jnp.sum(xr, axis=0)
    ps2 = jnp.sum(xr * xr, axis=0)

    @pl.when(pl.program_id(0) == 0)
    def _init():
        s_ref[...] = jnp.zeros_like(s_ref)
        s2_ref[...] = jnp.zeros_like(s2_ref)

    s_ref[...] += ps
    s2_ref[...] += ps2


def _mlp_body_alias(he_ref, g_ref, xp_ref, w1_ref, w2_ref, b2_ref, w3_ref,
                    b3_ref, x_ref, s_ref, s2_ref):
    del xp_ref
    _mlp_body(he_ref, g_ref, w1_ref, w2_ref, b2_ref, w3_ref, b3_ref,
              x_ref, s_ref, s2_ref)


def _mlp_pass(h_E, g, x_prev, w1_t, w2_t, b12, w3_t, b13, half):
    off = half * TILES_PER_SPLIT
    he_spec = pl.BlockSpec((EDGE_TILE, H), lambda i: (i + off, 0))
    g_spec = pl.BlockSpec((EDGE_TILE, H), lambda i: (i, 0))
    full = pl.BlockSpec((H, H), lambda i: (0, 0))
    vec = pl.BlockSpec((1, H), lambda i: (0, 0))
    stat_spec = pl.BlockSpec((8, H), lambda i: (0, 0))
    out_shape = (
        jax.ShapeDtypeStruct((N_EDGES, H), jnp.bfloat16),
        jax.ShapeDtypeStruct((8, H), jnp.float32),
        jax.ShapeDtypeStruct((8, H), jnp.float32),
    )
    out_specs = (
        pl.BlockSpec((EDGE_TILE, H), lambda i: (i + off, 0)),
        stat_spec,
        stat_spec,
    )
    if x_prev is None:
        return pl.pallas_call(
            _mlp_body,
            grid=(TILES_PER_SPLIT,),
            in_specs=[he_spec, g_spec, full, full, vec, full, vec],
            out_specs=out_specs,
            out_shape=out_shape,
        )(h_E, g, w1_t, w2_t, b12, w3_t, b13)
    return pl.pallas_call(
        _mlp_body_alias,
        grid=(TILES_PER_SPLIT,),
        in_specs=[
            he_spec, g_spec,
            pl.BlockSpec(memory_space=pltpu.MemorySpace.HBM),
            full, full, vec, full, vec,
        ],
        out_specs=out_specs,
        out_shape=out_shape,
        input_output_aliases={2: 0},
    )(h_E, g, x_prev, w1_t, w2_t, b12, w3_t, b13)


def _bn_body(x_ref, s_ref, s2_ref, g_ref, be_ref, o_ref):
    s = jnp.sum(s_ref[...], axis=0, keepdims=True)
    s2 = jnp.sum(s2_ref[...], axis=0, keepdims=True)
    inv_n = 1.0 / N_EDGES
    mean = s * inv_n
    var = s2 * inv_n - mean * mean
    inv = lax.rsqrt(var + BN_EPS)
    scale = g_ref[...] * inv
    shift = be_ref[...] - mean * scale
    o_ref[...] = x_ref[...].astype(jnp.float32) * scale + shift


def _bn_apply(x, s_all, s2_all, gamma, beta):
    n_tiles = N_EDGES // EDGE_TILE
    edge_spec = pl.BlockSpec((EDGE_TILE, H), lambda i: (i, 0))
    stat_spec = pl.BlockSpec((8 * NSPLIT, H), lambda i: (0, 0))
    return pl.pallas_call(
        _bn_body,
        grid=(n_tiles,),
        in_specs=[
            edge_spec,
            stat_spec, stat_spec,
            pl.BlockSpec((1, H), lambda i: (0, 0)),
            pl.BlockSpec((1, H), lambda i: (0, 0)),
        ],
        out_specs=edge_spec,
        out_shape=jax.ShapeDtypeStruct((N_EDGES, H), jnp.float32),
    )(x, s_all, s2_all, gamma, beta)


def kernel(h_V, h_E, edge_idx, batch_id, W11_w, W11_b, W12_w, W12_b,
           W13_w, W13_b, bn_gamma, bn_beta):
    del batch_id
    src = edge_idx[0].astype(jnp.int32)
    dst = edge_idx[1].astype(jnp.int32)

    wa_t = W11_w[:, :H].T
    w1_t = W11_w[:, H:2 * H].T.astype(jnp.bfloat16)
    wc_t = W11_w[:, 2 * H:].T
    b11 = W11_b.reshape(1, H)
    w2_t = W12_w.T.astype(jnp.bfloat16)
    w3_t = W13_w.T.astype(jnp.bfloat16)
    b12 = W12_b.reshape(1, H)
    b13 = W13_b.reshape(1, H)
    gamma = bn_gamma.reshape(1, H)
    beta = bn_beta.reshape(1, H)

    a, b = _node_proj(h_V, wa_t, wc_t, b11)
    gs = [_sc_gather(a, b, src, dst, k * E_SPLIT) for k in range(NSPLIT)]
    x = None
    stats, stats2 = [], []
    for k in range(NSPLIT):
        x, s_k, s2_k = _mlp_pass(h_E, gs[k], x, w1_t, w2_t, b12, w3_t, b13, k)
        stats.append(s_k)
        stats2.append(s2_k)
    s_all = jnp.concatenate(stats, axis=0)
    s2_all = jnp.concatenate(stats2, axis=0)
    return _bn_apply(x, s_all, s2_all, gamma, beta)


# speedup vs baseline: 1.0525x; 1.0086x over previous
import functools

import jax
import jax.numpy as jnp
from jax import lax
from jax.experimental import pallas as pl
from jax.experimental.pallas import tpu as pltpu
from jax.experimental.pallas import tpu_sc as plsc

N_NODES = 10000
N_EDGES = 320000
H = 128
BN_EPS = 1e-5

NUM_WORKERS = 32
NSPLIT = 1
E_SPLIT = N_EDGES // NSPLIT
EDGES_PER_W = E_SPLIT // NUM_WORKERS
CHUNK = 80
EDGE_TILE = 16000
TILES_PER_SPLIT = E_SPLIT // EDGE_TILE


def _gelu(x):
    return 0.5 * x * (1.0 + lax.erf(x * 0.7071067811865476))


def _node_proj_body(hv_ref, wa_ref, wc_ref, b_ref, a_ref, c_ref):
    hv = hv_ref[...]
    a_ref[...] = (
        jnp.dot(hv, wa_ref[...], preferred_element_type=jnp.float32) + b_ref[...]
    )
    c_ref[...] = jnp.dot(hv, wc_ref[...], preferred_element_type=jnp.float32)


def _node_proj(h_V, wa_t, wc_t, b11):
    return pl.pallas_call(
        _node_proj_body,
        out_shape=(
            jax.ShapeDtypeStruct((N_NODES, H), jnp.float32),
            jax.ShapeDtypeStruct((N_NODES, H), jnp.float32),
        ),
    )(h_V, wa_t, wc_t, b11)


NBANK = 3
CHUNKS_PER_W = EDGES_PER_W // CHUNK
HP = H // 2


def _gather_body(edge_base, a_hbm, b_hbm, src_hbm, dst_hbm, g_hbm,
                 idxs_v, idxd_v, rA0, rB0, rA1, rB1, rA2, rB2,
                 sg0, sg1, sg2, sw0, sw1, sw2):
    c = lax.axis_index("c")
    s = lax.axis_index("s")
    wid = s * 2 + c
    base = wid * EDGES_PER_W

    pltpu.sync_copy(src_hbm.at[pl.ds(edge_base + base, EDGES_PER_W)], idxs_v)
    pltpu.sync_copy(dst_hbm.at[pl.ds(edge_base + base, EDGES_PER_W)], idxd_v)

    banks = ((rA0, rB0, sg0, sw0), (rA1, rB1, sg1, sw1), (rA2, rB2, sg2, sw2))
    dummy = g_hbm.at[pl.ds(0, CHUNK)]

    def issue_gather(chunk, bank_i):
        rA, rB, sg, _ = banks[bank_i]
        off = pl.multiple_of(chunk * CHUNK, 8)
        pltpu.async_copy(a_hbm.at[idxs_v.at[pl.ds(off, CHUNK)]], rA, sg)
        pltpu.async_copy(b_hbm.at[idxd_v.at[pl.ds(off, CHUNK)]], rB, sg)

    issue_gather(0, 0)

    def make_branch(bank_i):
        rA, rB, sg, sw = banks[bank_i]
        nbank_i = (bank_i + 1) % NBANK
        _, _, _, sw_n = banks[nbank_i]

        def branch(g):
            @pl.when(g >= 2)
            def _():
                pltpu.make_async_copy(dummy, banks[nbank_i][0], sw_n).wait()

            @pl.when(g < CHUNKS_PER_W - 1)
            def _():
                issue_gather(g + 1, nbank_i)

            pltpu.make_async_copy(dummy, rA, sg).wait()
            pltpu.make_async_copy(dummy, rB, sg).wait()

            def row(r, carry):
                for k in range(H // 16):
                    sl = pl.ds(k * 16, 16)
                    rA[r, sl] = rA[r, sl] + rB[r, sl]
                return carry

            lax.fori_loop(0, CHUNK, row, 0)
            pltpu.async_copy(rA, g_hbm.at[pl.ds(base + g * CHUNK, CHUNK)], sw)

        return branch

    brs = [make_branch(i) for i in range(NBANK)]

    def step(g, carry):
        lax.switch(lax.rem(g, NBANK), brs, g)
        return carry

    lax.fori_loop(0, CHUNKS_PER_W, step, 0)

    for chunk in (CHUNKS_PER_W - 2, CHUNKS_PER_W - 1):
        rA, _, _, sw = banks[chunk % NBANK]
        pltpu.make_async_copy(dummy, rA, sw).wait()


def _sc_gather(a, b, src, dst, edge_base):
    mesh = plsc.VectorSubcoreMesh(
        core_axis_name="c", subcore_axis_name="s", num_cores=2, num_subcores=16
    )
    rows = pltpu.VMEM((CHUNK, H), jnp.float32)
    return pl.kernel(
        functools.partial(_gather_body, edge_base),
        out_type=jax.ShapeDtypeStruct((E_SPLIT, H), jnp.float32),
        mesh=mesh,
        scratch_types=[
            pltpu.VMEM((EDGES_PER_W,), jnp.int32),
            pltpu.VMEM((EDGES_PER_W,), jnp.int32),
            rows, rows, rows, rows, rows, rows,
            pltpu.SemaphoreType.DMA, pltpu.SemaphoreType.DMA,
            pltpu.SemaphoreType.DMA, pltpu.SemaphoreType.DMA,
            pltpu.SemaphoreType.DMA, pltpu.SemaphoreType.DMA,
        ],
    )(a, b, src, dst)


def _mlp_body(he_ref, g_ref, w1_ref, w2_ref, b2_ref, w3_ref, b3_ref,
              x_ref, s_ref, s2_ref):
    he = he_ref[...]
    pre1 = (
        jnp.dot(
            he.astype(jnp.bfloat16), w1_ref[...],
            preferred_element_type=jnp.float32,
        )
        + g_ref[...]
    )
    h1 = _gelu(pre1)
    h2 = _gelu(
        jnp.dot(
            h1.astype(jnp.bfloat16), w2_ref[...],
            preferred_element_type=jnp.float32,
        )
        + b2_ref[...]
    )
    msg = (
        jnp.dot(
            h2.astype(jnp.bfloat16), w3_ref[...],
            preferred_element_type=jnp.float32,
        )
        + b3_ref[...]
    )
    x = he + msg
    x_ref[...] = x.astype(jnp.bfloat16)

    xr = x.reshape(EDGE_TILE // 8, 8, H)
    ps = jnp.sum(xr, axis=0)
    ps2 = jnp.sum(xr * xr, axis=0)

    @pl.when(pl.program_id(0) == 0)
    def _init():
        s_ref[...] = jnp.zeros_like(s_ref)
        s2_ref[...] = jnp.zeros_like(s2_ref)

    s_ref[...] += ps
    s2_ref[...] += ps2


def _mlp_body_alias(he_ref, g_ref, xp_ref, w1_ref, w2_ref, b2_ref, w3_ref,
                    b3_ref, x_ref, s_ref, s2_ref):
    del xp_ref
    _mlp_body(he_ref, g_ref, w1_ref, w2_ref, b2_ref, w3_ref, b3_ref,
              x_ref, s_ref, s2_ref)


def _mlp_pass(h_E, g, x_prev, w1_t, w2_t, b12, w3_t, b13, half):
    off = half * TILES_PER_SPLIT
    he_spec = pl.BlockSpec((EDGE_TILE, H), lambda i: (i + off, 0))
    g_spec = pl.BlockSpec((EDGE_TILE, H), lambda i: (i, 0))
    full = pl.BlockSpec((H, H), lambda i: (0, 0))
    vec = pl.BlockSpec((1, H), lambda i: (0, 0))
    stat_spec = pl.BlockSpec((8, H), lambda i: (0, 0))
    out_shape = (
        jax.ShapeDtypeStruct((N_EDGES, H), jnp.bfloat16),
        jax.ShapeDtypeStruct((8, H), jnp.float32),
        jax.ShapeDtypeStruct((8, H), jnp.float32),
    )
    out_specs = (
        pl.BlockSpec((EDGE_TILE, H), lambda i: (i + off, 0)),
        stat_spec,
        stat_spec,
    )
    if x_prev is None:
        return pl.pallas_call(
            _mlp_body,
            grid=(TILES_PER_SPLIT,),
            in_specs=[he_spec, g_spec, full, full, vec, full, vec],
            out_specs=out_specs,
            out_shape=out_shape,
        )(h_E, g, w1_t, w2_t, b12, w3_t, b13)
    return pl.pallas_call(
        _mlp_body_alias,
        grid=(TILES_PER_SPLIT,),
        in_specs=[
            he_spec, g_spec,
            pl.BlockSpec(memory_space=pltpu.MemorySpace.HBM),
            full, full, vec, full, vec,
        ],
        out_specs=out_specs,
        out_shape=out_shape,
        input_output_aliases={2: 0},
    )(h_E, g, x_prev, w1_t, w2_t, b12, w3_t, b13)


def _bn_body(x_ref, s_ref, s2_ref, g_ref, be_ref, o_ref):
    s = jnp.sum(s_ref[...], axis=0, keepdims=True)
    s2 = jnp.sum(s2_ref[...], axis=0, keepdims=True)
    inv_n = 1.0 / N_EDGES
    mean = s * inv_n
    var = s2 * inv_n - mean * mean
    inv = lax.rsqrt(var + BN_EPS)
    scale = g_ref[...] * inv
    shift = be_ref[...] - mean * scale
    o_ref[...] = x_ref[...].astype(jnp.float32) * scale + shift


def _bn_apply(x, s_all, s2_all, gamma, beta):
    n_tiles = N_EDGES // EDGE_TILE
    edge_spec = pl.BlockSpec((EDGE_TILE, H), lambda i: (i, 0))
    stat_spec = pl.BlockSpec((8 * NSPLIT, H), lambda i: (0, 0))
    return pl.pallas_call(
        _bn_body,
        grid=(n_tiles,),
        in_specs=[
            edge_spec,
            stat_spec, stat_spec,
            pl.BlockSpec((1, H), lambda i: (0, 0)),
            pl.BlockSpec((1, H), lambda i: (0, 0)),
        ],
        out_specs=edge_spec,
        out_shape=jax.ShapeDtypeStruct((N_EDGES, H), jnp.float32),
    )(x, s_all, s2_all, gamma, beta)


def kernel(h_V, h_E, edge_idx, batch_id, W11_w, W11_b, W12_w, W12_b,
           W13_w, W13_b, bn_gamma, bn_beta):
    del batch_id
    src = edge_idx[0].astype(jnp.int32)
    dst = edge_idx[1].astype(jnp.int32)

    wa_t = W11_w[:, :H].T
    w1_t = W11_w[:, H:2 * H].T.astype(jnp.bfloat16)
    wc_t = W11_w[:, 2 * H:].T
    b11 = W11_b.reshape(1, H)
    w2_t = W12_w.T.astype(jnp.bfloat16)
    w3_t = W13_w.T.astype(jnp.bfloat16)
    b12 = W12_b.reshape(1, H)
    b13 = W13_b.reshape(1, H)
    gamma = bn_gamma.reshape(1, H)
    beta = bn_beta.reshape(1, H)

    a, b = _node_proj(h_V, wa_t, wc_t, b11)
    gs = [_sc_gather(a, b, src, dst, k * E_SPLIT) for k in range(NSPLIT)]
    x = None
    stats, stats2 = [], []
    for k in range(NSPLIT):
        x, s_k, s2_k = _mlp_pass(h_E, gs[k], x, w1_t, w2_t, b12, w3_t, b13, k)
        stats.append(s_k)
        stats2.append(s2_k)
    s_all = jnp.concatenate(stats, axis=0)
    s2_all = jnp.concatenate(stats2, axis=0)
    return _bn_apply(x, s_all, s2_all, gamma, beta)


# speedup vs baseline: 1.0594x; 1.0065x over previous
import functools

import jax
import jax.numpy as jnp
from jax import lax
from jax.experimental import pallas as pl
from jax.experimental.pallas import tpu as pltpu
from jax.experimental.pallas import tpu_sc as plsc

N_NODES = 10000
N_EDGES = 320000
H = 128
BN_EPS = 1e-5

NUM_WORKERS = 32
NSPLIT = 1
E_SPLIT = N_EDGES // NSPLIT
EDGES_PER_W = E_SPLIT // NUM_WORKERS
CHUNK = 80
EDGE_TILE = 16000
TILES_PER_SPLIT = E_SPLIT // EDGE_TILE


def _gelu(x):
    return 0.5 * x * (1.0 + lax.erf(x * 0.7071067811865476))


def _node_proj_body(hv_ref, wa_ref, wc_ref, b_ref, a_ref, c_ref):
    hv = hv_ref[...]
    a_ref[...] = (
        jnp.dot(hv, wa_ref[...], preferred_element_type=jnp.float32) + b_ref[...]
    )
    c_ref[...] = jnp.dot(hv, wc_ref[...], preferred_element_type=jnp.float32)


def _node_proj(h_V, wa_t, wc_t, b11):
    return pl.pallas_call(
        _node_proj_body,
        out_shape=(
            jax.ShapeDtypeStruct((N_NODES, H), jnp.float32),
            jax.ShapeDtypeStruct((N_NODES, H), jnp.float32),
        ),
    )(h_V, wa_t, wc_t, b11)


NBANK = 3
CHUNKS_PER_W = EDGES_PER_W // CHUNK
HP = H // 2


def _gather_body(edge_base, a_hbm, b_hbm, src_hbm, dst_hbm, g_hbm,
                 idxs_v, idxd_v, rA0, rB0, rA1, rB1, rA2, rB2,
                 sg0, sg1, sg2, sw0, sw1, sw2):
    c = lax.axis_index("c")
    s = lax.axis_index("s")
    wid = s * 2 + c
    base = wid * EDGES_PER_W

    pltpu.sync_copy(src_hbm.at[pl.ds(edge_base + base, EDGES_PER_W)], idxs_v)
    pltpu.sync_copy(dst_hbm.at[pl.ds(edge_base + base, EDGES_PER_W)], idxd_v)

    banks = ((rA0, rB0, sg0, sw0), (rA1, rB1, sg1, sw1), (rA2, rB2, sg2, sw2))
    dummy = g_hbm.at[pl.ds(0, CHUNK)]

    def issue_gather(chunk, bank_i):
        rA, rB, sg, _ = banks[bank_i]
        off = pl.multiple_of(chunk * CHUNK, 8)
        pltpu.async_copy(a_hbm.at[idxs_v.at[pl.ds(off, CHUNK)]], rA, sg)
        pltpu.async_copy(b_hbm.at[idxd_v.at[pl.ds(off, CHUNK)]], rB, sg)

    issue_gather(0, 0)

    def make_branch(bank_i):
        rA, rB, sg, sw = banks[bank_i]
        nbank_i = (bank_i + 1) % NBANK
        _, _, _, sw_n = banks[nbank_i]

        def branch(g):
            @pl.when(g >= 2)
            def _():
                pltpu.make_async_copy(dummy, banks[nbank_i][0], sw_n).wait()

            @pl.when(g < CHUNKS_PER_W - 1)
            def _():
                issue_gather(g + 1, nbank_i)

            pltpu.make_async_copy(dummy, rA, sg).wait()
            pltpu.make_async_copy(dummy, rB, sg).wait()

            def row(r, carry):
                for k in range(H // 16):
                    sl = pl.ds(k * 16, 16)
                    rA[r, sl] = rA[r, sl] + rB[r, sl]
                return carry

            lax.fori_loop(0, CHUNK, row, 0)
            pltpu.async_copy(rA, g_hbm.at[pl.ds(base + g * CHUNK, CHUNK)], sw)

        return branch

    brs = [make_branch(i) for i in range(NBANK)]

    def step(g, carry):
        lax.switch(lax.rem(g, NBANK), brs, g)
        return carry

    lax.fori_loop(0, CHUNKS_PER_W, step, 0)

    for chunk in (CHUNKS_PER_W - 2, CHUNKS_PER_W - 1):
        rA, _, _, sw = banks[chunk % NBANK]
        pltpu.make_async_copy(dummy, rA, sw).wait()


def _sc_gather(a, b, src, dst, edge_base):
    mesh = plsc.VectorSubcoreMesh(
        core_axis_name="c", subcore_axis_name="s", num_cores=2, num_subcores=16
    )
    rows = pltpu.VMEM((CHUNK, H), jnp.float32)
    return pl.kernel(
        functools.partial(_gather_body, edge_base),
        out_type=jax.ShapeDtypeStruct((E_SPLIT, H), jnp.float32),
        mesh=mesh,
        scratch_types=[
            pltpu.VMEM((EDGES_PER_W,), jnp.int32),
            pltpu.VMEM((EDGES_PER_W,), jnp.int32),
            rows, rows, rows, rows, rows, rows,
            pltpu.SemaphoreType.DMA, pltpu.SemaphoreType.DMA,
            pltpu.SemaphoreType.DMA, pltpu.SemaphoreType.DMA,
            pltpu.SemaphoreType.DMA, pltpu.SemaphoreType.DMA,
        ],
    )(a, b, src, dst)


def _mlp_body(he_ref, g_ref, w1_ref, w2_ref, b2_ref, w3_ref, b3_ref,
              x_ref, s_ref, s2_ref):
    he = he_ref[...]
    pre1 = (
        jnp.dot(
            he.astype(jnp.bfloat16), w1_ref[...],
            preferred_element_type=jnp.float32,
        )
        + g_ref[...]
    )
    h1 = _gelu(pre1)
    h2 = _gelu(
        jnp.dot(
            h1.astype(jnp.bfloat16), w2_ref[...],
            preferred_element_type=jnp.float32,
        )
        + b2_ref[...]
    )
    msg = (
        jnp.dot(
            h2.astype(jnp.bfloat16), w3_ref[...],
            preferred_element_type=jnp.float32,
        )
        + b3_ref[...]
    )
    x = he + msg
    x_ref[...] = x.astype(jnp.bfloat16)

    xr = x.reshape(EDGE_TILE // 8, 8, H)
    ps = jnp.sum(xr, axis=0)
    ps2 = jnp.sum(xr * xr, axis=0)

    @pl.when(pl.program_id(0) == 0)
    def _init():
        s_ref[...] = jnp.zeros_like(s_ref)
        s2_ref[...] = jnp.zeros_like(s2_ref)

    s_ref[...] += ps
    s2_ref[...] += ps2


def _mlp_body_alias(he_ref, g_ref, xp_ref, w1_ref, w2_ref, b2_ref, w3_ref,
                    b3_ref, x_ref, s_ref, s2_ref):
    del xp_ref
    _mlp_body(he_ref, g_ref, w1_ref, w2_ref, b2_ref, w3_ref, b3_ref,
              x_ref, s_ref, s2_ref)


def _mlp_pass(h_E, g, x_prev, w1_t, w2_t, b12, w3_t, b13, half):
    off = half * TILES_PER_SPLIT
    he_spec = pl.BlockSpec((EDGE_TILE, H), lambda i: (i + off, 0))
    g_spec = pl.BlockSpec((EDGE_TILE, H), lambda i: (i, 0))
    full = pl.BlockSpec((H, H), lambda i: (0, 0))
    vec = pl.BlockSpec((1, H), lambda i: (0, 0))
    stat_spec = pl.BlockSpec((8, H), lambda i: (0, 0))
    out_shape = (
        jax.ShapeDtypeStruct((N_EDGES, H), jnp.bfloat16),
        jax.ShapeDtypeStruct((8, H), jnp.float32),
        jax.ShapeDtypeStruct((8, H), jnp.float32),
    )
    out_specs = (
        pl.BlockSpec((EDGE_TILE, H), lambda i: (i + off, 0)),
        stat_spec,
        stat_spec,
    )
    if x_prev is None:
        return pl.pallas_call(
            _mlp_body,
            grid=(TILES_PER_SPLIT,),
            in_specs=[he_spec, g_spec, full, full, vec, full, vec],
            out_specs=out_specs,
            out_shape=out_shape,
        )(h_E, g, w1_t, w2_t, b12, w3_t, b13)
    return pl.pallas_call(
        _mlp_body_alias,
        grid=(TILES_PER_SPLIT,),
        in_specs=[
            he_spec, g_spec,
            pl.BlockSpec(memory_space=pltpu.MemorySpace.HBM),
            full, full, vec, full, vec,
        ],
        out_specs=out_specs,
        out_shape=out_shape,
        input_output_aliases={2: 0},
    )(h_E, g, x_prev, w1_t, w2_t, b12, w3_t, b13)


BN_TILE = 32000


def _bn_body(x_ref, s_ref, s2_ref, g_ref, be_ref, o_ref):
    s = jnp.sum(s_ref[...], axis=0, keepdims=True)
    s2 = jnp.sum(s2_ref[...], axis=0, keepdims=True)
    inv_n = 1.0 / N_EDGES
    mean = s * inv_n
    var = s2 * inv_n - mean * mean
    inv = lax.rsqrt(var + BN_EPS)
    scale = g_ref[...] * inv
    shift = be_ref[...] - mean * scale
    o_ref[...] = x_ref[...].astype(jnp.float32) * scale + shift


def _bn_apply(x, s_all, s2_all, gamma, beta):
    n_tiles = N_EDGES // BN_TILE
    edge_spec = pl.BlockSpec((BN_TILE, H), lambda i: (i, 0))
    stat_spec = pl.BlockSpec((8 * NSPLIT, H), lambda i: (0, 0))
    return pl.pallas_call(
        _bn_body,
        grid=(n_tiles,),
        in_specs=[
            edge_spec,
            stat_spec, stat_spec,
            pl.BlockSpec((1, H), lambda i: (0, 0)),
            pl.BlockSpec((1, H), lambda i: (0, 0)),
        ],
        out_specs=edge_spec,
        out_shape=jax.ShapeDtypeStruct((N_EDGES, H), jnp.float32),
    )(x, s_all, s2_all, gamma, beta)


def kernel(h_V, h_E, edge_idx, batch_id, W11_w, W11_b, W12_w, W12_b,
           W13_w, W13_b, bn_gamma, bn_beta):
    del batch_id
    src = edge_idx[0].astype(jnp.int32)
    dst = edge_idx[1].astype(jnp.int32)

    wa_t = W11_w[:, :H].T
    w1_t = W11_w[:, H:2 * H].T.astype(jnp.bfloat16)
    wc_t = W11_w[:, 2 * H:].T
    b11 = W11_b.reshape(1, H)
    w2_t = W12_w.T.astype(jnp.bfloat16)
    w3_t = W13_w.T.astype(jnp.bfloat16)
    b12 = W12_b.reshape(1, H)
    b13 = W13_b.reshape(1, H)
    gamma = bn_gamma.reshape(1, H)
    beta = bn_beta.reshape(1, H)

    a, b = _node_proj(h_V, wa_t, wc_t, b11)
    gs = [_sc_gather(a, b, src, dst, k * E_SPLIT) for k in range(NSPLIT)]
    x = None
    stats, stats2 = [], []
    for k in range(NSPLIT):
        x, s_k, s2_k = _mlp_pass(h_E, gs[k], x, w1_t, w2_t, b12, w3_t, b13, k)
        stats.append(s_k)
        stats2.append(s2_k)
    s_all = jnp.concatenate(stats, axis=0)
    s2_all = jnp.concatenate(stats2, axis=0)
    return _bn_apply(x, s_all, s2_all, gamma, beta)
